# baseline (speedup 1.0000x reference)
import functools

import jax
import jax.numpy as jnp
from jax import lax
from jax.experimental import pallas as pl
from jax.experimental.pallas import tpu as pltpu
from jax.experimental.pallas import tpu_sc as plsc

NUM_CORES = 2
NUM_SUBCORES = 16
NW = NUM_CORES * NUM_SUBCORES
CH = 16
NSEM = 4


@functools.partial(jax.jit, static_argnums=(2, 3))
def _embed(idx2, table, per_w, hidden):
    mesh = plsc.VectorSubcoreMesh(core_axis_name="c", subcore_axis_name="s")
    n_ch = per_w // CH

    @functools.partial(
        pl.kernel,
        out_type=jax.ShapeDtypeStruct((NW, per_w, hidden), jnp.float32),
        mesh=mesh,
        scratch_types=[
            pltpu.VMEM((per_w,), jnp.int32),
            pltpu.VMEM_SHARED((NUM_SUBCORES, per_w, hidden), jnp.float32),
            [pltpu.SemaphoreType.DMA] * NSEM,
        ],
    )
    def body(idx_hbm, table_hbm, out_hbm, idx_s, rows_sp, sems):
        wid = lax.axis_index("s") * NUM_CORES + lax.axis_index("c")
        sid = lax.axis_index("s")
        pltpu.sync_copy(idx_hbm.at[wid], idx_s)
        mine = rows_sp.at[sid]

        def issue(c, k):
            base = c * CH
            vec = idx_s[pl.ds(base, CH)]
            for j in range(CH):
                r = vec[j]
                pltpu.async_copy(
                    table_hbm.at[pl.ds(r, 1)], mine.at[pl.ds(base + j, 1)], sems[k]
                )

        def drain(k):
            pltpu.make_async_copy(
                table_hbm.at[pl.ds(0, CH)], mine.at[pl.ds(0, CH)], sems[k]
            ).wait()

        for c in range(n_ch):
            issue(c, c % NSEM)
            if c >= NSEM:
                drain((c - NSEM) % NSEM)
        for c in range(n_ch - NSEM, n_ch):
            drain(c % NSEM)
        pltpu.sync_copy(mine, out_hbm.at[wid])

    return body(idx2, table)


def kernel(labels, train, dropout_prob, table):
    del train, dropout_prob
    batch = labels.shape[0]
    per_w = batch // NW
    idx2 = labels.astype(jnp.int32).reshape(NW, per_w)
    out = _embed(idx2, table, per_w, table.shape[1])
    return out.reshape(batch, table.shape[1])
